# baseline (speedup 1.0000x reference)
import dataclasses
import functools

import jax
import jax.numpy as jnp
from jax import lax
from jax.experimental import pallas as pl
from jax.experimental.pallas import tpu as pltpu
from jax.experimental.pallas import tpu_sc as plsc

_USER_NUM = 6000
_ITEM_NUM = 4000
_N = _USER_NUM + _ITEM_NUM
_D = 128
_E = 320000
_LAYERS = 3

_NC = 2
_NS = 16
_NWORK = _NC * _NS
_LANES = 16
_W = 128
_NWIN0 = 48
_NWIN1 = 112
_NWINT = _NS * (_NWIN0 + _NWIN1)
_EPAD = _NWINT * _W
_NPAD = 10240
_ROWS_PER_TILE = _NPAD // _NS

_mesh = plsc.VectorSubcoreMesh(
    core_axis_name="c", subcore_axis_name="s", num_cores=_NC, num_subcores=_NS
)

_sc_params = pltpu.CompilerParams()
if "needs_layout_passes" in pltpu.CompilerParams.__dataclass_fields__:
    _sc_params = dataclasses.replace(_sc_params, needs_layout_passes=False)


def _sc_layer(emb, pk0, pk1):
    @functools.partial(
        pl.kernel,
        out_type=jax.ShapeDtypeStruct((_NC, _NPAD, _D), jnp.float32),
        mesh=_mesh,
        compiler_params=_sc_params,
        scratch_types=[
            pltpu.VMEM((3, _W), jnp.int32),
            pltpu.VMEM((3, _W), jnp.int32),
            pltpu.VMEM((3, _W), jnp.int32),
            pltpu.VMEM((3, _W), jnp.int32),
            pltpu.VMEM((_W, _D), jnp.float32),
            pltpu.VMEM((_W, _D), jnp.float32),
            pltpu.VMEM_SHARED((_NPAD, _D), jnp.float32),
            pltpu.SemaphoreType.DMA,
            pltpu.SemaphoreType.DMA,
            pltpu.SemaphoreType.DMA,
            pltpu.SemaphoreType.DMA,
            pltpu.SemaphoreType.DMA,
            pltpu.SemaphoreType.DMA,
            pltpu.SemaphoreType.DMA,
            pltpu.SemaphoreType.DMA,
        ],
    )
    def layer(emb_hbm, pk0_hbm, pk1_hbm, out_hbm,
              pka, pkb, pkc, pkd, rows0, rows1, acc_sh,
              sg0, sg1, ss0, ss1, sp0, sp1, sp2, sp3):
        c = lax.axis_index("c")
        s = lax.axis_index("s")
        rows = (rows0, rows1)
        pk = (pka, pkb, pkc, pkd)
        sem_g = (sg0, sg1)
        sem_s = (ss0, ss1)
        sem_p = (sp0, sp1, sp2, sp3)

        @pl.loop(0, _W)
        def _zero_rows(r):
            for c8 in range(_D // _LANES):
                rows0[r, pl.ds(c8 * _LANES, _LANES)] = jnp.zeros(
                    (_LANES,), jnp.float32)

        base = s * _ROWS_PER_TILE
        for k in range(_ROWS_PER_TILE // _W):
            pltpu.sync_copy(rows0.at[pl.ds(0, _W)],
                            acc_sh.at[pl.ds(base + k * _W, _W)])
        plsc.subcore_barrier()

        def scale(buf, pkq):
            vref = pkq.at[1]

            @pl.loop(0, _W, unroll=4)
            def _scale(r):
                vv = plsc.bitcast(
                    plsc.load_gather(vref, [jnp.full((_LANES,), r, jnp.int32)]),
                    jnp.float32)
                for c8 in range(_D // _LANES):
                    sl = pl.ds(c8 * _LANES, _LANES)
                    buf[r, sl] = buf[r, sl] * vv

        def run(nwin, pk_hbm):
            pltpu.async_copy(pk_hbm.at[s, 0], pk[0], sem_p[0])
            pltpu.async_copy(pk_hbm.at[s, 1], pk[1], sem_p[1])
            pltpu.async_copy(pk_hbm.at[s, 2], pk[2], sem_p[2])
            pltpu.make_async_copy(pk_hbm.at[s, 0], pk[0], sem_p[0]).wait()
            pltpu.async_copy(emb_hbm.at[pk[0].at[2]], rows[0], sem_g[0])

            def phase(win, q, wait_scatter, issue_pk, issue_gather):
                b = q % 2
                nb = (b + 1) % 2
                if wait_scatter:
                    pltpu.make_async_copy(rows[nb], acc_sh.at[pk[q].at[0]],
                                          sem_s[nb]).wait()
                if issue_pk:
                    pltpu.async_copy(pk_hbm.at[s, win + 3], pk[(q + 3) % 4],
                                     sem_p[(q + 3) % 4])
                if issue_gather:
                    nq = (q + 1) % 4
                    pltpu.make_async_copy(pk_hbm.at[s, win + 1], pk[nq],
                                          sem_p[nq]).wait()
                    pltpu.async_copy(emb_hbm.at[pk[nq].at[2]], rows[nb],
                                     sem_g[nb])
                pltpu.make_async_copy(emb_hbm.at[pk[q].at[2]], rows[b],
                                      sem_g[b]).wait()
                scale(rows[b], pk[q])
                pltpu.async_copy(rows[b], acc_sh.at[pk[q].at[0]], sem_s[b],
                                 add=True)

            phase(0, 0, False, True, True)
            phase(1, 1, True, True, True)
            phase(2, 2, True, True, True)
            phase(3, 3, True, True, True)

            @pl.loop(4, nwin - 4, step=4)
            def _window(j):
                phase(j, 0, True, True, True)
                phase(j + 1, 1, True, True, True)
                phase(j + 2, 2, True, True, True)
                phase(j + 3, 3, True, True, True)

            phase(nwin - 4, 0, True, True, True)
            phase(nwin - 3, 1, True, False, True)
            phase(nwin - 2, 2, True, False, True)
            phase(nwin - 1, 3, True, False, False)

            pltpu.make_async_copy(rows[1], acc_sh.at[pk[3].at[0]],
                                  sem_s[1]).wait()

        @pl.when(c == 0)
        def _core0():
            run(_NWIN0, pk0_hbm)

        @pl.when(c == 1)
        def _core1():
            run(_NWIN1, pk1_hbm)

        plsc.subcore_barrier()

        for k in range(_ROWS_PER_TILE // _W):
            pltpu.sync_copy(acc_sh.at[pl.ds(base + k * _W, _W)],
                            out_hbm.at[c, pl.ds(base + k * _W, _W)])

    return layer(emb, pk0, pk1)


def _combine(partials, total_prev):
    def body(p_ref, t_ref, emb_ref, tot_ref):
        e = p_ref[0] + p_ref[1]
        emb_ref[...] = e
        tot_ref[...] = t_ref[...] + e

    return pl.pallas_call(
        body,
        out_shape=(jax.ShapeDtypeStruct((_NPAD, _D), jnp.float32),
                   jax.ShapeDtypeStruct((_NPAD, _D), jnp.float32)),
    )(partials, total_prev)


def _finalize(partials, total_prev):
    def body(p_ref, t_ref, o_ref):
        o_ref[...] = (t_ref[...] + p_ref[0] + p_ref[1]) * 0.25

    return pl.pallas_call(
        body,
        out_shape=jax.ShapeDtypeStruct((_NPAD, _D), jnp.float32),
    )(partials, total_prev)


def kernel(edge_index, edge_vals, user_embeds, item_embeds, keep_rate):
    del keep_rate
    emb0 = jnp.concatenate(
        [user_embeds, item_embeds,
         jnp.zeros((_NPAD - _N, _D), jnp.float32)], axis=0)
    dst = edge_index[0]
    src = edge_index[1]
    pad = _EPAD - _E
    dst_w = jnp.pad(dst, (0, pad)).reshape(_NWINT, _W)
    src_w = jnp.pad(src, (0, pad)).reshape(_NWINT, _W)
    val_bits = lax.bitcast_convert_type(
        jnp.pad(edge_vals, (0, pad)), jnp.int32).reshape(_NWINT, _W)
    pk_all = jnp.stack([dst_w, val_bits, src_w], axis=1)
    n0 = _NS * _NWIN0
    pk0 = pk_all[:n0].reshape(_NS, _NWIN0, 3, _W)
    pk1 = pk_all[n0:].reshape(_NS, _NWIN1, 3, _W)

    total = emb0
    emb = emb0
    for layer in range(_LAYERS):
        p = _sc_layer(emb, pk0, pk1)
        if layer < _LAYERS - 1:
            emb, total = _combine(p, total)
        else:
            total = _finalize(p, total)
    return total[:_USER_NUM], total[_USER_NUM:_N]


# speedup vs baseline: 1.0517x; 1.0517x over previous
import dataclasses
import functools

import jax
import jax.numpy as jnp
from jax import lax
from jax.experimental import pallas as pl
from jax.experimental.pallas import tpu as pltpu
from jax.experimental.pallas import tpu_sc as plsc

_USER_NUM = 6000
_ITEM_NUM = 4000
_N = _USER_NUM + _ITEM_NUM
_D = 128
_E = 320000
_LAYERS = 3

_NC = 2
_NS = 16
_NWORK = _NC * _NS
_LANES = 16
_W = 128
_NWIN = -(-(_E // _NWORK) // _W)
_EPAD = _NWORK * _NWIN * _W
_NPAD = 10240
_ROWS_PER_TILE = _NPAD // _NS

_mesh = plsc.VectorSubcoreMesh(
    core_axis_name="c", subcore_axis_name="s", num_cores=_NC, num_subcores=_NS
)

_sc_params = pltpu.CompilerParams()
if "needs_layout_passes" in pltpu.CompilerParams.__dataclass_fields__:
    _sc_params = dataclasses.replace(_sc_params, needs_layout_passes=False)


def _sc_layer(emb, src_w, dst_w, val_w):
    @functools.partial(
        pl.kernel,
        out_type=jax.ShapeDtypeStruct((_NC, _NPAD, _D), jnp.float32),
        mesh=_mesh,
        compiler_params=_sc_params,
        scratch_types=[
            pltpu.VMEM((_NWIN, _W), jnp.int32),
            pltpu.VMEM((_NWIN, _W), jnp.int32),
            pltpu.VMEM((_NWIN, _W), jnp.float32),
            pltpu.VMEM((_W, _D), jnp.float32),
            pltpu.VMEM_SHARED((_NPAD, _D), jnp.float32),
        ],
    )
    def layer(emb_hbm, src_hbm, dst_hbm, val_hbm, out_hbm,
              src_v, dst_v, val_v, rows_v, acc_sh):
        c = lax.axis_index("c")
        s = lax.axis_index("s")
        w = c * _NS + s

        pltpu.sync_copy(src_hbm.at[w], src_v)
        pltpu.sync_copy(dst_hbm.at[w], dst_v)
        pltpu.sync_copy(val_hbm.at[w], val_v)

        @pl.loop(0, _W)
        def _zero_rows(r):
            for c8 in range(_D // _LANES):
                rows_v[r, pl.ds(c8 * _LANES, _LANES)] = jnp.zeros(
                    (_LANES,), jnp.float32)

        base = s * _ROWS_PER_TILE
        for k in range(_ROWS_PER_TILE // _W):
            pltpu.sync_copy(rows_v.at[pl.ds(0, _W)],
                            acc_sh.at[pl.ds(base + k * _W, _W)])
        plsc.subcore_barrier()

        @pl.loop(0, _NWIN)
        def _window(j):
            pltpu.sync_copy(emb_hbm.at[src_v.at[j]], rows_v)

            @pl.loop(0, _W)
            def _scale(r):
                vv = plsc.load_gather(
                    val_v,
                    [jnp.full((_LANES,), j, jnp.int32),
                     jnp.full((_LANES,), r, jnp.int32)],
                )
                for c8 in range(_D // _LANES):
                    sl = pl.ds(c8 * _LANES, _LANES)
                    rows_v[r, sl] = rows_v[r, sl] * vv

            pltpu.sync_copy(rows_v, acc_sh.at[dst_v.at[j]], add=True)

        plsc.subcore_barrier()

        for k in range(_ROWS_PER_TILE // _W):
            pltpu.sync_copy(acc_sh.at[pl.ds(base + k * _W, _W)],
                            out_hbm.at[c, pl.ds(base + k * _W, _W)])

    return layer(emb, src_w, dst_w, val_w)


def _combine(partials, total_prev):
    def body(p_ref, t_ref, emb_ref, tot_ref):
        e = p_ref[0] + p_ref[1]
        emb_ref[...] = e
        tot_ref[...] = t_ref[...] + e

    return pl.pallas_call(
        body,
        out_shape=(jax.ShapeDtypeStruct((_NPAD, _D), jnp.float32),
                   jax.ShapeDtypeStruct((_NPAD, _D), jnp.float32)),
    )(partials, total_prev)


def _finalize(partials, total_prev):
    def body(p_ref, t_ref, o_ref):
        o_ref[...] = (t_ref[...] + p_ref[0] + p_ref[1]) * 0.25

    return pl.pallas_call(
        body,
        out_shape=jax.ShapeDtypeStruct((_NPAD, _D), jnp.float32),
    )(partials, total_prev)


def kernel(edge_index, edge_vals, user_embeds, item_embeds, keep_rate):
    del keep_rate
    emb0 = jnp.concatenate(
        [user_embeds, item_embeds,
         jnp.zeros((_NPAD - _N, _D), jnp.float32)], axis=0)
    dst = edge_index[0]
    src = edge_index[1]
    pad = _EPAD - _E
    src_w = jnp.pad(src, (0, pad)).reshape(_NWORK, _NWIN, _W)
    dst_w = jnp.pad(dst, (0, pad)).reshape(_NWORK, _NWIN, _W)
    val_w = jnp.pad(edge_vals, (0, pad)).reshape(_NWORK, _NWIN, _W)

    total = emb0
    emb = emb0
    for layer in range(_LAYERS):
        p = _sc_layer(emb, src_w, dst_w, val_w)
        if layer < _LAYERS - 1:
            emb, total = _combine(p, total)
        else:
            total = _finalize(p, total)
    return total[:_USER_NUM], total[_USER_NUM:_N]
